# Initial kernel scaffold; baseline (speedup 1.0000x reference)
import jax
import jax.numpy as jnp
from jax.experimental import pallas as pl


def kernel(x, data_str, edge_index, lins0_w, lins0_b, lins1_w, lins1_b, lin11_w, lin11_b, lin3_w, lin3_b, convs_w1, convs_w2, convs1_w1, convs1_w2):
    raise NotImplementedError("write your pallas kernel here")


# speedup vs baseline: 17.0990x; 17.0990x over previous
import functools

import numpy as np
import jax
import jax.numpy as jnp
from jax import lax
from jax.experimental import pallas as pl
from jax.experimental.pallas import tpu as pltpu
from jax.experimental.pallas import tpu_sc as plsc

_N = 10000
_E = 320000
_DF = 128
_DS = 58
_H = 64
_ALPHA = 0.4
_THETA = 0.9

_NC = 2
_NS = 16
_NW = _NC * _NS
_K = 125
_CH = _E // (_NW * _K)
_RPS = 640
_TAIL = _N - 15 * _RPS
_ZR = 128

_RB = 2000


def _mesh():
    return plsc.VectorSubcoreMesh(core_axis_name="c", subcore_axis_name="s")


def _zero_slab(s, sp_ref, zbuf):
    lo = s * _RPS
    for t in range(_RPS // _ZR):
        @pl.when(jnp.logical_or(s < 15, t < _TAIL // _ZR))
        def _():
            pltpu.sync_copy(zbuf, sp_ref.at[pl.ds(lo + t * _ZR, _ZR)])

    @pl.when(s == 15)
    def _():
        pltpu.sync_copy(zbuf.at[pl.ds(0, _TAIL % _ZR)],
                        sp_ref.at[pl.ds(15 * _RPS + (_TAIL // _ZR) * _ZR,
                                        _TAIL % _ZR)])


def _dump(c, s, sp_ref, out):
    @pl.when(s < 15)
    def _():
        pltpu.sync_copy(sp_ref.at[pl.ds(s * _RPS, _RPS)],
                        out.at[c, pl.ds(s * _RPS, _RPS)])

    @pl.when(s == 15)
    def _():
        pltpu.sync_copy(sp_ref.at[pl.ds(15 * _RPS, _TAIL)],
                        out.at[c, pl.ds(15 * _RPS, _TAIL)])


def _deg_body(col2d, degp, col_v, ones_v, zbuf, deg_sp):
    c = lax.axis_index("c")
    s = lax.axis_index("s")
    w = c * _NS + s

    def fill_ones(i, carry):
        ones_v[i, :] = jnp.ones((16,), jnp.float32)
        return carry

    lax.fori_loop(0, _K, fill_ones, 0)

    def fill_z(i, carry):
        zbuf[i, :] = jnp.zeros((16,), jnp.float32)
        return carry

    lax.fori_loop(0, _ZR, fill_z, 0)

    _zero_slab(s, deg_sp, zbuf)
    pltpu.sync_copy(col2d.at[pl.ds(w * _CH, _CH)], col_v)
    plsc.subcore_barrier()

    def chunk(j, carry):
        pltpu.sync_copy(ones_v, deg_sp.at[col_v.at[j]], add=True)
        return carry

    lax.fori_loop(0, _CH, chunk, 0)
    plsc.subcore_barrier()
    _dump(c, s, deg_sp, degp)


_sc_deg = pl.kernel(
    _deg_body,
    out_type=jax.ShapeDtypeStruct((_NC, _N, 16), jnp.float32),
    mesh=_mesh(),
    scratch_types=[
        pltpu.VMEM((_CH, _K), jnp.int32),
        pltpu.VMEM((_K, 16), jnp.float32),
        pltpu.VMEM((_ZR, 16), jnp.float32),
        pltpu.VMEM_SHARED((_N, 16), jnp.float32),
    ],
    compiler_params=pltpu.CompilerParams(use_tc_tiling_on_sc=False),
    name="sc_deg_hist",
)


def _mp_body(h, row2d, col2d, out, row_v, col_v, rb, zbuf, agg_sp):
    c = lax.axis_index("c")
    s = lax.axis_index("s")
    w = c * _NS + s

    def fill_z(i, carry):
        for t in range(4):
            zbuf[i, pl.ds(16 * t, 16)] = jnp.zeros((16,), jnp.float32)
        return carry

    lax.fori_loop(0, _ZR, fill_z, 0)
    _zero_slab(s, agg_sp, zbuf)
    pltpu.sync_copy(row2d.at[pl.ds(w * _CH, _CH)], row_v)
    pltpu.sync_copy(col2d.at[pl.ds(w * _CH, _CH)], col_v)
    plsc.subcore_barrier()

    def chunk(j, carry):
        pltpu.sync_copy(h.at[row_v.at[j]], rb)
        pltpu.sync_copy(rb, agg_sp.at[col_v.at[j]], add=True)
        return carry

    lax.fori_loop(0, _CH, chunk, 0)
    plsc.subcore_barrier()
    _dump(c, s, agg_sp, out)


_sc_mp = pl.kernel(
    _mp_body,
    out_type=jax.ShapeDtypeStruct((_NC, _N, _H), jnp.float32),
    mesh=_mesh(),
    scratch_types=[
        pltpu.VMEM((_CH, _K), jnp.int32),
        pltpu.VMEM((_CH, _K), jnp.int32),
        pltpu.VMEM((_K, _H), jnp.float32),
        pltpu.VMEM((_ZR, _H), jnp.float32),
        pltpu.VMEM_SHARED((_N, _H), jnp.float32),
    ],
    compiler_params=pltpu.CompilerParams(use_tc_tiling_on_sc=False),
    name="sc_mp_round",
)


def _pre_body(degp, x, dstr, w0, b0, w11, b11, h, hs, h1, hs1, dis):
    deg = degp[0, :, 0:1] + degp[1, :, 0:1]
    d = jnp.where(deg > 0, lax.rsqrt(jnp.maximum(deg, 1e-12)), 0.0)
    a = jnp.maximum(
        jnp.dot(x[...], w0[...], preferred_element_type=jnp.float32) + b0[...], 0.0)
    b = jnp.maximum(
        jnp.dot(dstr[...], w11[...], preferred_element_type=jnp.float32) + b11[...], 0.0)
    h[...] = a
    hs[...] = d * a
    h1[...] = b
    hs1[...] = d * b
    dis[...] = d


_pre = pl.pallas_call(
    _pre_body,
    grid=(_N // _RB,),
    in_specs=[
        pl.BlockSpec((_NC, _RB, 16), lambda i: (0, i, 0)),
        pl.BlockSpec((_RB, _DF), lambda i: (i, 0)),
        pl.BlockSpec((_RB, _DS), lambda i: (i, 0)),
        pl.BlockSpec((_DF, _H), lambda i: (0, 0)),
        pl.BlockSpec((1, _H), lambda i: (0, 0)),
        pl.BlockSpec((_DS, _H), lambda i: (0, 0)),
        pl.BlockSpec((1, _H), lambda i: (0, 0)),
    ],
    out_specs=[
        pl.BlockSpec((_RB, _H), lambda i: (i, 0)),
        pl.BlockSpec((_RB, _H), lambda i: (i, 0)),
        pl.BlockSpec((_RB, _H), lambda i: (i, 0)),
        pl.BlockSpec((_RB, _H), lambda i: (i, 0)),
        pl.BlockSpec((_RB, 1), lambda i: (i, 0)),
    ],
    out_shape=[
        jax.ShapeDtypeStruct((_N, _H), jnp.float32),
        jax.ShapeDtypeStruct((_N, _H), jnp.float32),
        jax.ShapeDtypeStruct((_N, _H), jnp.float32),
        jax.ShapeDtypeStruct((_N, _H), jnp.float32),
        jax.ShapeDtypeStruct((_N, 1), jnp.float32),
    ],
    name="tc_pre",
)


def _mix(beta, part, dis, h0, w1, w2):
    t = part[0] + part[1]
    d = dis[...]
    aggh = (1.0 - _ALPHA) * (d * t)
    h0a = _ALPHA * h0[...]
    o = (1.0 - beta) * aggh + beta * jnp.dot(
        aggh, w1[...], preferred_element_type=jnp.float32)
    o = o + (1.0 - beta) * h0a + beta * jnp.dot(
        h0a, w2[...], preferred_element_type=jnp.float32)
    return jnp.maximum(o, 0.0), d


def _comb_mid_body(beta, part, dis, h0, w1, w2, hn, hsn):
    o, d = _mix(beta, part, dis, h0, w1, w2)
    hn[...] = o
    hsn[...] = d * o


def _comb_last_body(beta, part, dis, h0, w1, w2, lw, lb, z):
    o, _ = _mix(beta, part, dis, h0, w1, w2)
    z[...] = jnp.dot(o, lw[...], preferred_element_type=jnp.float32) + lb[...]


def _make_comb_mid(beta):
    return pl.pallas_call(
        functools.partial(_comb_mid_body, beta),
        grid=(_N // _RB,),
        in_specs=[
            pl.BlockSpec((_NC, _RB, _H), lambda i: (0, i, 0)),
            pl.BlockSpec((_RB, 1), lambda i: (i, 0)),
            pl.BlockSpec((_RB, _H), lambda i: (i, 0)),
            pl.BlockSpec((_H, _H), lambda i: (0, 0)),
            pl.BlockSpec((_H, _H), lambda i: (0, 0)),
        ],
        out_specs=[
            pl.BlockSpec((_RB, _H), lambda i: (i, 0)),
            pl.BlockSpec((_RB, _H), lambda i: (i, 0)),
        ],
        out_shape=[
            jax.ShapeDtypeStruct((_N, _H), jnp.float32),
            jax.ShapeDtypeStruct((_N, _H), jnp.float32),
        ],
        name="tc_combine_mid",
    )


def _make_comb_last(beta):
    return pl.pallas_call(
        functools.partial(_comb_last_body, beta),
        grid=(_N // _RB,),
        in_specs=[
            pl.BlockSpec((_NC, _RB, _H), lambda i: (0, i, 0)),
            pl.BlockSpec((_RB, 1), lambda i: (i, 0)),
            pl.BlockSpec((_RB, _H), lambda i: (i, 0)),
            pl.BlockSpec((_H, _H), lambda i: (0, 0)),
            pl.BlockSpec((_H, _H), lambda i: (0, 0)),
            pl.BlockSpec((_H, 1), lambda i: (0, 0)),
            pl.BlockSpec((1, 1), lambda i: (0, 0)),
        ],
        out_specs=[pl.BlockSpec((_RB, 1), lambda i: (i, 0))],
        out_shape=[jax.ShapeDtypeStruct((_N, 1), jnp.float32)],
        name="tc_combine_last",
    )


_BETA0 = float(np.log(_THETA / 1.0 + 1.0))
_BETA1 = float(np.log(_THETA / 2.0 + 1.0))
_comb_mid0 = _make_comb_mid(_BETA0)
_comb_last1 = _make_comb_last(_BETA1)


def kernel(x, data_str, edge_index, lins0_w, lins0_b, lins1_w, lins1_b,
           lin11_w, lin11_b, lin3_w, lin3_b,
           convs_w1, convs_w2, convs1_w1, convs1_w2):
    row2d = edge_index[0].reshape(_E // _K, _K)
    col2d = edge_index[1].reshape(_E // _K, _K)

    degp = _sc_deg(col2d)
    h, hs, h1, hs1, dis = _pre(degp, x, data_str,
                               lins0_w, lins0_b.reshape(1, _H),
                               lin11_w, lin11_b.reshape(1, _H))

    p = _sc_mp(hs, row2d, col2d)
    _, hsL = _comb_mid0(p, dis, h, convs_w1[0], convs_w2[0])
    p = _sc_mp(hsL, row2d, col2d)
    (z,) = _comb_last1(p, dis, h, convs_w1[1], convs_w2[1],
                       lins1_w, lins1_b.reshape(1, 1))

    p = _sc_mp(hs1, row2d, col2d)
    _, hs1L = _comb_mid0(p, dis, h1, convs1_w1[0], convs1_w2[0])
    p = _sc_mp(hs1L, row2d, col2d)
    (z1,) = _comb_last1(p, dis, h1, convs1_w1[1], convs1_w2[1],
                        lin3_w, lin3_b.reshape(1, 1))

    return (z, z1)


# speedup vs baseline: 20.8615x; 1.2200x over previous
import functools

import numpy as np
import jax
import jax.numpy as jnp
from jax import lax
from jax.experimental import pallas as pl
from jax.experimental.pallas import tpu as pltpu
from jax.experimental.pallas import tpu_sc as plsc

_N = 10000
_E = 320000
_DF = 128
_DS = 58
_H = 64
_ALPHA = 0.4
_THETA = 0.9

_NC = 2
_NS = 16
_NW = _NC * _NS
_K = 125
_NCHUNK = _E // _K
_CPT = _NCHUNK // _NS
_CHD = _NCHUNK // _NW
_RPS = 640
_TAIL = _N - 15 * _RPS
_ZR = 128

_RB = 2000


def _mesh():
    return plsc.VectorSubcoreMesh(core_axis_name="c", subcore_axis_name="s")


def _zero_slab(s, sp_ref, zbuf):
    lo = s * _RPS
    for t in range(_RPS // _ZR):
        @pl.when(jnp.logical_or(s < 15, t < _TAIL // _ZR))
        def _():
            pltpu.sync_copy(zbuf, sp_ref.at[pl.ds(lo + t * _ZR, _ZR)])

    @pl.when(s == 15)
    def _():
        pltpu.sync_copy(zbuf.at[pl.ds(0, _TAIL % _ZR)],
                        sp_ref.at[pl.ds(15 * _RPS + (_TAIL // _ZR) * _ZR,
                                        _TAIL % _ZR)])


def _dump(c, s, sp_ref, out):
    @pl.when(s < 15)
    def _():
        pltpu.sync_copy(sp_ref.at[pl.ds(s * _RPS, _RPS)],
                        out.at[c, pl.ds(s * _RPS, _RPS)])

    @pl.when(s == 15)
    def _():
        pltpu.sync_copy(sp_ref.at[pl.ds(15 * _RPS, _TAIL)],
                        out.at[c, pl.ds(15 * _RPS, _TAIL)])


def _deg_body(col2d, degp, col_v, ones_v, zbuf, deg_sp):
    c = lax.axis_index("c")
    s = lax.axis_index("s")
    w = c * _NS + s

    def fill_ones(i, carry):
        ones_v[i, :] = jnp.ones((16,), jnp.float32)
        return carry

    lax.fori_loop(0, _K, fill_ones, 0)

    def fill_z(i, carry):
        zbuf[i, :] = jnp.zeros((16,), jnp.float32)
        return carry

    lax.fori_loop(0, _ZR, fill_z, 0)

    _zero_slab(s, deg_sp, zbuf)
    pltpu.sync_copy(col2d.at[pl.ds(w * _CHD, _CHD)], col_v)
    plsc.subcore_barrier()

    def chunk(j, carry):
        pltpu.sync_copy(ones_v, deg_sp.at[col_v.at[j]], add=True)
        return carry

    lax.fori_loop(0, _CHD, chunk, 0)
    plsc.subcore_barrier()
    _dump(c, s, deg_sp, degp)


_sc_deg = pl.kernel(
    _deg_body,
    out_type=jax.ShapeDtypeStruct((_NC, _N, 16), jnp.float32),
    mesh=_mesh(),
    scratch_types=[
        pltpu.VMEM((_CHD, _K), jnp.int32),
        pltpu.VMEM((_K, 16), jnp.float32),
        pltpu.VMEM((_ZR, 16), jnp.float32),
        pltpu.VMEM_SHARED((_N, 16), jnp.float32),
    ],
    compiler_params=pltpu.CompilerParams(use_tc_tiling_on_sc=False),
    name="sc_deg_hist",
)


def _mp_body(h2, row2d, col2d, out, row_v, col_v, rba, rbb, zbuf, agg_sp,
             gsa, gsb, ssa, ssb):
    c = lax.axis_index("c")
    s = lax.axis_index("s")

    def fill_z(i, carry):
        for t in range(_H // 16):
            zbuf[i, pl.ds(16 * t, 16)] = jnp.zeros((16,), jnp.float32)
        return carry

    lax.fori_loop(0, _ZR, fill_z, 0)
    _zero_slab(s, agg_sp, zbuf)
    pltpu.sync_copy(row2d.at[pl.ds(s * _CPT, _CPT)], row_v)
    pltpu.sync_copy(col2d.at[pl.ds(s * _CPT, _CPT)], col_v)
    plsc.subcore_barrier()

    hb = h2.at[c]

    pltpu.async_copy(hb.at[row_v.at[0]], rba, gsa)

    def chunk(i, carry):
        j0 = 2 * i
        pltpu.make_async_copy(hb.at[row_v.at[j0]], rba, gsa).wait()

        @pl.when(i > 0)
        def _():
            pltpu.make_async_copy(rbb, agg_sp.at[col_v.at[j0 - 1]], ssb).wait()

        pltpu.async_copy(hb.at[row_v.at[j0 + 1]], rbb, gsb)
        pltpu.async_copy(rba, agg_sp.at[col_v.at[j0]], ssa, add=True)
        pltpu.make_async_copy(hb.at[row_v.at[j0 + 1]], rbb, gsb).wait()
        pltpu.make_async_copy(rba, agg_sp.at[col_v.at[j0]], ssa).wait()

        @pl.when(i < _CPT // 2 - 1)
        def _():
            pltpu.async_copy(hb.at[row_v.at[j0 + 2]], rba, gsa)

        pltpu.async_copy(rbb, agg_sp.at[col_v.at[j0 + 1]], ssb, add=True)
        return carry

    lax.fori_loop(0, _CPT // 2, chunk, 0)
    pltpu.make_async_copy(rbb, agg_sp.at[col_v.at[_CPT - 1]], ssb).wait()
    plsc.subcore_barrier()
    _dump(c, s, agg_sp, out)


_sc_mp = pl.kernel(
    _mp_body,
    out_type=jax.ShapeDtypeStruct((_NC, _N, _H), jnp.float32),
    mesh=_mesh(),
    scratch_types=[
        pltpu.VMEM((_CPT, _K), jnp.int32),
        pltpu.VMEM((_CPT, _K), jnp.int32),
        pltpu.VMEM((_K, _H), jnp.float32),
        pltpu.VMEM((_K, _H), jnp.float32),
        pltpu.VMEM((_ZR, _H), jnp.float32),
        pltpu.VMEM_SHARED((_N, _H), jnp.float32),
        pltpu.SemaphoreType.DMA,
        pltpu.SemaphoreType.DMA,
        pltpu.SemaphoreType.DMA,
        pltpu.SemaphoreType.DMA,
    ],
    compiler_params=pltpu.CompilerParams(use_tc_tiling_on_sc=False),
    name="sc_mp_round",
)


def _pre_body(degp, x, dstr, w0, b0, w11, b11, h, h1, hsb, dis):
    deg = degp[0, :, 0:1] + degp[1, :, 0:1]
    d = jnp.where(deg > 0, lax.rsqrt(jnp.maximum(deg, 1e-12)), 0.0)
    a = jnp.maximum(
        jnp.dot(x[...], w0[...], preferred_element_type=jnp.float32) + b0[...], 0.0)
    b = jnp.maximum(
        jnp.dot(dstr[...], w11[...], preferred_element_type=jnp.float32) + b11[...], 0.0)
    h[...] = a
    h1[...] = b
    hsb[...] = d * jnp.stack([a, b], axis=0)
    dis[...] = d


_pre = pl.pallas_call(
    _pre_body,
    grid=(_N // _RB,),
    in_specs=[
        pl.BlockSpec((_NC, _RB, 16), lambda i: (0, i, 0)),
        pl.BlockSpec((_RB, _DF), lambda i: (i, 0)),
        pl.BlockSpec((_RB, _DS), lambda i: (i, 0)),
        pl.BlockSpec((_DF, _H), lambda i: (0, 0)),
        pl.BlockSpec((1, _H), lambda i: (0, 0)),
        pl.BlockSpec((_DS, _H), lambda i: (0, 0)),
        pl.BlockSpec((1, _H), lambda i: (0, 0)),
    ],
    out_specs=[
        pl.BlockSpec((_RB, _H), lambda i: (i, 0)),
        pl.BlockSpec((_RB, _H), lambda i: (i, 0)),
        pl.BlockSpec((_NC, _RB, _H), lambda i: (0, i, 0)),
        pl.BlockSpec((_RB, 1), lambda i: (i, 0)),
    ],
    out_shape=[
        jax.ShapeDtypeStruct((_N, _H), jnp.float32),
        jax.ShapeDtypeStruct((_N, _H), jnp.float32),
        jax.ShapeDtypeStruct((_NC, _N, _H), jnp.float32),
        jax.ShapeDtypeStruct((_N, 1), jnp.float32),
    ],
    name="tc_pre",
)


def _mix(beta, agg2, dis, h0, h10, w1a, w2a, w1b, w2b):
    d = dis[...]
    oa = _mix_half(beta, (1.0 - _ALPHA) * (d * agg2[0]), _ALPHA * h0[...], w1a, w2a)
    ob = _mix_half(beta, (1.0 - _ALPHA) * (d * agg2[1]), _ALPHA * h10[...], w1b, w2b)
    return oa, ob, d


def _mix_half(beta, aggh, h0a, w1, w2):
    o = (1.0 - beta) * aggh + beta * jnp.dot(
        aggh, w1[...], preferred_element_type=jnp.float32)
    o = o + (1.0 - beta) * h0a + beta * jnp.dot(
        h0a, w2[...], preferred_element_type=jnp.float32)
    return jnp.maximum(o, 0.0)


def _comb_mid_body(beta, agg2, dis, h0, h10, w1a, w2a, w1b, w2b, hsb):
    oa, ob, d = _mix(beta, agg2, dis, h0, h10, w1a, w2a, w1b, w2b)
    hsb[...] = d * jnp.stack([oa, ob], axis=0)


def _comb_last_body(beta, agg2, dis, h0, h10, w1a, w2a, w1b, w2b,
                    lwa, lba, lwb, lbb, z, z1):
    oa, ob, _ = _mix(beta, agg2, dis, h0, h10, w1a, w2a, w1b, w2b)
    z[...] = jnp.dot(oa, lwa[...], preferred_element_type=jnp.float32) + lba[...]
    z1[...] = jnp.dot(ob, lwb[...], preferred_element_type=jnp.float32) + lbb[...]


_W_SPECS = [
    pl.BlockSpec((_NC, _RB, _H), lambda i: (0, i, 0)),
    pl.BlockSpec((_RB, 1), lambda i: (i, 0)),
    pl.BlockSpec((_RB, _H), lambda i: (i, 0)),
    pl.BlockSpec((_RB, _H), lambda i: (i, 0)),
    pl.BlockSpec((_H, _H), lambda i: (0, 0)),
    pl.BlockSpec((_H, _H), lambda i: (0, 0)),
    pl.BlockSpec((_H, _H), lambda i: (0, 0)),
    pl.BlockSpec((_H, _H), lambda i: (0, 0)),
]

_comb_mid0 = pl.pallas_call(
    functools.partial(_comb_mid_body, float(np.log(_THETA / 1.0 + 1.0))),
    grid=(_N // _RB,),
    in_specs=_W_SPECS,
    out_specs=[pl.BlockSpec((_NC, _RB, _H), lambda i: (0, i, 0))],
    out_shape=[jax.ShapeDtypeStruct((_NC, _N, _H), jnp.float32)],
    name="tc_combine_mid",
)

_comb_last1 = pl.pallas_call(
    functools.partial(_comb_last_body, float(np.log(_THETA / 2.0 + 1.0))),
    grid=(_N // _RB,),
    in_specs=_W_SPECS + [
        pl.BlockSpec((_H, 1), lambda i: (0, 0)),
        pl.BlockSpec((1, 1), lambda i: (0, 0)),
        pl.BlockSpec((_H, 1), lambda i: (0, 0)),
        pl.BlockSpec((1, 1), lambda i: (0, 0)),
    ],
    out_specs=[
        pl.BlockSpec((_RB, 1), lambda i: (i, 0)),
        pl.BlockSpec((_RB, 1), lambda i: (i, 0)),
    ],
    out_shape=[
        jax.ShapeDtypeStruct((_N, 1), jnp.float32),
        jax.ShapeDtypeStruct((_N, 1), jnp.float32),
    ],
    name="tc_combine_last",
)


def kernel(x, data_str, edge_index, lins0_w, lins0_b, lins1_w, lins1_b,
           lin11_w, lin11_b, lin3_w, lin3_b,
           convs_w1, convs_w2, convs1_w1, convs1_w2):
    row2d = edge_index[0].reshape(_NCHUNK, _K)
    col2d = edge_index[1].reshape(_NCHUNK, _K)

    degp = _sc_deg(col2d)
    h, h1, hsb, dis = _pre(degp, x, data_str,
                           lins0_w, lins0_b.reshape(1, _H),
                           lin11_w, lin11_b.reshape(1, _H))

    p = _sc_mp(hsb, row2d, col2d)
    (hsb,) = _comb_mid0(p, dis, h, h1,
                        convs_w1[0], convs_w2[0], convs1_w1[0], convs1_w2[0])
    p = _sc_mp(hsb, row2d, col2d)
    z, z1 = _comb_last1(p, dis, h, h1,
                        convs_w1[1], convs_w2[1], convs1_w1[1], convs1_w2[1],
                        lins1_w, lins1_b.reshape(1, 1),
                        lin3_w, lin3_b.reshape(1, 1))
    return (z, z1)


# speedup vs baseline: 22.6655x; 1.0865x over previous
import functools

import numpy as np
import jax
import jax.numpy as jnp
from jax import lax
from jax.experimental import pallas as pl
from jax.experimental.pallas import tpu as pltpu
from jax.experimental.pallas import tpu_sc as plsc

_N = 10000
_N2 = _N // 2
_E = 320000
_DF = 128
_DS = 58
_H = 64
_ALPHA = 0.4
_THETA = 0.9

_NC = 2
_NS = 16
_NW = _NC * _NS
_K = 125
_NCHUNK = _E // _K
_CPT = _NCHUNK // _NS
_CHD = _NCHUNK // _NW
_RPS = 640
_TAIL = _N - 15 * _RPS
_ZR = 128

_RB = 1000


def _mesh():
    return plsc.VectorSubcoreMesh(core_axis_name="c", subcore_axis_name="s")


def _zero_slab(s, sp_ref, zbuf):
    lo = s * _RPS
    for t in range(_RPS // _ZR):
        @pl.when(jnp.logical_or(s < 15, t < _TAIL // _ZR))
        def _():
            pltpu.sync_copy(zbuf, sp_ref.at[pl.ds(lo + t * _ZR, _ZR)])

    @pl.when(s == 15)
    def _():
        pltpu.sync_copy(zbuf.at[pl.ds(0, _TAIL % _ZR)],
                        sp_ref.at[pl.ds(15 * _RPS + (_TAIL // _ZR) * _ZR,
                                        _TAIL % _ZR)])


def _dump(c, s, sp_ref, out):
    @pl.when(s < 15)
    def _():
        pltpu.sync_copy(sp_ref.at[pl.ds(s * _RPS, _RPS)],
                        out.at[c, pl.ds(s * _RPS, _RPS)])

    @pl.when(s == 15)
    def _():
        pltpu.sync_copy(sp_ref.at[pl.ds(15 * _RPS, _TAIL)],
                        out.at[c, pl.ds(15 * _RPS, _TAIL)])


def _deg_body(col2d, degp, col_v, ones_v, zbuf, deg_sp):
    c = lax.axis_index("c")
    s = lax.axis_index("s")
    w = c * _NS + s

    def fill_ones(i, carry):
        for t in range(_H // 16):
            ones_v[i, pl.ds(16 * t, 16)] = jnp.ones((16,), jnp.float32)
        return carry

    lax.fori_loop(0, _K, fill_ones, 0)

    def fill_z(i, carry):
        for t in range(_H // 16):
            zbuf[i, pl.ds(16 * t, 16)] = jnp.zeros((16,), jnp.float32)
        return carry

    lax.fori_loop(0, _ZR, fill_z, 0)

    _zero_slab(s, deg_sp, zbuf)
    pltpu.sync_copy(col2d.at[pl.ds(w * _CHD, _CHD)], col_v)
    plsc.subcore_barrier()

    def chunk(j, carry):
        pltpu.sync_copy(ones_v, deg_sp.at[col_v.at[j]], add=True)
        return carry

    lax.fori_loop(0, _CHD, chunk, 0)
    plsc.subcore_barrier()
    _dump(c, s, deg_sp, degp)


_sc_deg = pl.kernel(
    _deg_body,
    out_type=jax.ShapeDtypeStruct((_NC, _N, _H), jnp.float32),
    mesh=_mesh(),
    scratch_types=[
        pltpu.VMEM((_CHD, _K), jnp.int32),
        pltpu.VMEM((_K, _H), jnp.float32),
        pltpu.VMEM((_ZR, _H), jnp.float32),
        pltpu.VMEM_SHARED((_N, _H), jnp.float32),
    ],
    compiler_params=pltpu.CompilerParams(use_tc_tiling_on_sc=False),
    name="sc_deg_hist",
)


def _mp_body(h2, row2d, col2d, out, row_v, col_v, rba, rbb, zbuf, agg_sp,
             gsa, gsb, ssa, ssb):
    c = lax.axis_index("c")
    s = lax.axis_index("s")

    def fill_z(i, carry):
        for t in range(_H // 16):
            zbuf[i, pl.ds(16 * t, 16)] = jnp.zeros((16,), jnp.float32)
        return carry

    lax.fori_loop(0, _ZR, fill_z, 0)
    _zero_slab(s, agg_sp, zbuf)
    pltpu.sync_copy(row2d.at[pl.ds(s * _CPT, _CPT)], row_v)
    pltpu.sync_copy(col2d.at[pl.ds(s * _CPT, _CPT)], col_v)
    plsc.subcore_barrier()

    hb = h2.at[c]

    pltpu.async_copy(hb.at[row_v.at[0]], rba, gsa)

    def chunk(i, carry):
        j0 = 2 * i
        pltpu.make_async_copy(hb.at[row_v.at[j0]], rba, gsa).wait()

        @pl.when(i > 0)
        def _():
            pltpu.make_async_copy(rbb, agg_sp.at[col_v.at[j0 - 1]], ssb).wait()

        pltpu.async_copy(hb.at[row_v.at[j0 + 1]], rbb, gsb)
        pltpu.async_copy(rba, agg_sp.at[col_v.at[j0]], ssa, add=True)
        pltpu.make_async_copy(hb.at[row_v.at[j0 + 1]], rbb, gsb).wait()
        pltpu.make_async_copy(rba, agg_sp.at[col_v.at[j0]], ssa).wait()

        @pl.when(i < _CPT // 2 - 1)
        def _():
            pltpu.async_copy(hb.at[row_v.at[j0 + 2]], rba, gsa)

        pltpu.async_copy(rbb, agg_sp.at[col_v.at[j0 + 1]], ssb, add=True)
        return carry

    lax.fori_loop(0, _CPT // 2, chunk, 0)
    pltpu.make_async_copy(rbb, agg_sp.at[col_v.at[_CPT - 1]], ssb).wait()
    plsc.subcore_barrier()
    _dump(c, s, agg_sp, out)


_sc_mp = pl.kernel(
    _mp_body,
    out_type=jax.ShapeDtypeStruct((_NC, _N, _H), jnp.float32),
    mesh=_mesh(),
    scratch_types=[
        pltpu.VMEM((_CPT, _K), jnp.int32),
        pltpu.VMEM((_CPT, _K), jnp.int32),
        pltpu.VMEM((_K, _H), jnp.float32),
        pltpu.VMEM((_K, _H), jnp.float32),
        pltpu.VMEM((_ZR, _H), jnp.float32),
        pltpu.VMEM_SHARED((_N, _H), jnp.float32),
        pltpu.SemaphoreType.DMA,
        pltpu.SemaphoreType.DMA,
        pltpu.SemaphoreType.DMA,
        pltpu.SemaphoreType.DMA,
    ],
    compiler_params=pltpu.CompilerParams(use_tc_tiling_on_sc=False),
    name="sc_mp_round",
)


def _pre_body(dgp, x2, ds2, w0b, b0b, w11b, b11b, h128, h1128, hsb, dis):
    deg = dgp[0] + dgp[1]
    d = jnp.where(deg > 0, lax.rsqrt(jnp.maximum(deg, 1e-12)), 0.0)
    a = jnp.maximum(
        jnp.dot(x2[...], w0b[...], preferred_element_type=jnp.float32) + b0b[...], 0.0)
    b = jnp.maximum(
        jnp.dot(ds2[...], w11b[...], preferred_element_type=jnp.float32) + b11b[...], 0.0)
    h128[...] = a
    h1128[...] = b
    hsb[...] = d * jnp.stack([a, b], axis=0)
    dis[...] = d


_pre = pl.pallas_call(
    _pre_body,
    grid=(_N2 // _RB,),
    in_specs=[
        pl.BlockSpec((_NC, _RB, 128), lambda i: (0, i, 0)),
        pl.BlockSpec((_RB, 2 * _DF), lambda i: (i, 0)),
        pl.BlockSpec((_RB, 128), lambda i: (i, 0)),
        pl.BlockSpec((2 * _DF, 128), lambda i: (0, 0)),
        pl.BlockSpec((1, 128), lambda i: (0, 0)),
        pl.BlockSpec((128, 128), lambda i: (0, 0)),
        pl.BlockSpec((1, 128), lambda i: (0, 0)),
    ],
    out_specs=[
        pl.BlockSpec((_RB, 128), lambda i: (i, 0)),
        pl.BlockSpec((_RB, 128), lambda i: (i, 0)),
        pl.BlockSpec((_NC, _RB, 128), lambda i: (0, i, 0)),
        pl.BlockSpec((_RB, 128), lambda i: (i, 0)),
    ],
    out_shape=[
        jax.ShapeDtypeStruct((_N2, 128), jnp.float32),
        jax.ShapeDtypeStruct((_N2, 128), jnp.float32),
        jax.ShapeDtypeStruct((_NC, _N2, 128), jnp.float32),
        jax.ShapeDtypeStruct((_N2, 128), jnp.float32),
    ],
    name="tc_pre",
)


def _mix(beta, p, dis, h0, h10, w1a, w2a, w1b, w2b):
    d = dis[...]
    oa = _mix_half(beta, (1.0 - _ALPHA) * (d * p[0]), _ALPHA * h0[...], w1a, w2a)
    ob = _mix_half(beta, (1.0 - _ALPHA) * (d * p[1]), _ALPHA * h10[...], w1b, w2b)
    return oa, ob, d


def _mix_half(beta, aggh, h0a, w1, w2):
    o = (1.0 - beta) * aggh + beta * jnp.dot(
        aggh, w1[...], preferred_element_type=jnp.float32)
    o = o + (1.0 - beta) * h0a + beta * jnp.dot(
        h0a, w2[...], preferred_element_type=jnp.float32)
    return jnp.maximum(o, 0.0)


def _comb_mid_body(beta, p, dis, h0, h10, w1a, w2a, w1b, w2b, hsb):
    oa, ob, d = _mix(beta, p, dis, h0, h10, w1a, w2a, w1b, w2b)
    hsb[...] = d * jnp.stack([oa, ob], axis=0)


def _comb_last_body(beta, p, dis, h0, h10, w1a, w2a, w1b, w2b,
                    lwa, lwb, lb2, z2):
    oa, ob, _ = _mix(beta, p, dis, h0, h10, w1a, w2a, w1b, w2b)
    za = jnp.dot(oa, lwa[...], preferred_element_type=jnp.float32)
    zb = jnp.dot(ob, lwb[...], preferred_element_type=jnp.float32)
    z2[...] = jnp.stack([za, zb], axis=0) + lb2[...]


_W_SPECS = [
    pl.BlockSpec((_NC, _RB, 128), lambda i: (0, i, 0)),
    pl.BlockSpec((_RB, 128), lambda i: (i, 0)),
    pl.BlockSpec((_RB, 128), lambda i: (i, 0)),
    pl.BlockSpec((_RB, 128), lambda i: (i, 0)),
    pl.BlockSpec((128, 128), lambda i: (0, 0)),
    pl.BlockSpec((128, 128), lambda i: (0, 0)),
    pl.BlockSpec((128, 128), lambda i: (0, 0)),
    pl.BlockSpec((128, 128), lambda i: (0, 0)),
]

_comb_mid0 = pl.pallas_call(
    functools.partial(_comb_mid_body, float(np.log(_THETA / 1.0 + 1.0))),
    grid=(_N2 // _RB,),
    in_specs=_W_SPECS,
    out_specs=[pl.BlockSpec((_NC, _RB, 128), lambda i: (0, i, 0))],
    out_shape=[jax.ShapeDtypeStruct((_NC, _N2, 128), jnp.float32)],
    name="tc_combine_mid",
)

_comb_last1 = pl.pallas_call(
    functools.partial(_comb_last_body, float(np.log(_THETA / 2.0 + 1.0))),
    grid=(_N2 // _RB,),
    in_specs=_W_SPECS + [
        pl.BlockSpec((128, 2), lambda i: (0, 0)),
        pl.BlockSpec((128, 2), lambda i: (0, 0)),
        pl.BlockSpec((_NC, 1, 2), lambda i: (0, 0, 0)),
    ],
    out_specs=[pl.BlockSpec((_NC, _RB, 2), lambda i: (0, i, 0))],
    out_shape=[jax.ShapeDtypeStruct((_NC, _N2, 2), jnp.float32)],
    name="tc_combine_last",
)


def _blkdiag(w):
    fi, fo = w.shape
    zz = jnp.zeros((fi, fo), jnp.float32)
    return jnp.concatenate([
        jnp.concatenate([w, zz], axis=1),
        jnp.concatenate([zz, w], axis=1),
    ], axis=0)


def kernel(x, data_str, edge_index, lins0_w, lins0_b, lins1_w, lins1_b,
           lin11_w, lin11_b, lin3_w, lin3_b,
           convs_w1, convs_w2, convs1_w1, convs1_w2):
    row2d = edge_index[0].reshape(_NCHUNK, _K)
    col2d = edge_index[1].reshape(_NCHUNK, _K)

    x2 = x.reshape(_N2, 2 * _DF)
    ds64 = jnp.pad(data_str, ((0, 0), (0, _H - _DS)))
    ds2 = ds64.reshape(_N2, 128)
    w11p = jnp.pad(lin11_w, ((0, _H - _DS), (0, 0)))
    b2 = jnp.concatenate([lins0_b, lins0_b]).reshape(1, 128)
    b112 = jnp.concatenate([lin11_b, lin11_b]).reshape(1, 128)

    degp = _sc_deg(col2d)
    h, h1, hsb, dis = _pre(degp.reshape(_NC, _N2, 128), x2, ds2,
                           _blkdiag(lins0_w), b2, _blkdiag(w11p), b112)

    p = _sc_mp(hsb.reshape(_NC, _N, _H), row2d, col2d)
    (hsb,) = _comb_mid0(p.reshape(_NC, _N2, 128), dis, h, h1,
                        _blkdiag(convs_w1[0]), _blkdiag(convs_w2[0]),
                        _blkdiag(convs1_w1[0]), _blkdiag(convs1_w2[0]))
    p = _sc_mp(hsb.reshape(_NC, _N, _H), row2d, col2d)
    lb2 = jnp.stack([jnp.broadcast_to(lins1_b, (2,)),
                     jnp.broadcast_to(lin3_b, (2,))]).reshape(_NC, 1, 2)
    (z2,) = _comb_last1(p.reshape(_NC, _N2, 128), dis, h, h1,
                        _blkdiag(convs_w1[1]), _blkdiag(convs_w2[1]),
                        _blkdiag(convs1_w1[1]), _blkdiag(convs1_w2[1]),
                        _blkdiag(lins1_w), _blkdiag(lin3_w), lb2)
    z = z2[0].reshape(_N, 1)
    z1 = z2[1].reshape(_N, 1)
    return (z, z1)


# speedup vs baseline: 22.7005x; 1.0015x over previous
import functools

import numpy as np
import jax
import jax.numpy as jnp
from jax import lax
from jax.experimental import pallas as pl
from jax.experimental.pallas import tpu as pltpu
from jax.experimental.pallas import tpu_sc as plsc

_N = 10000
_N2 = _N // 2
_E = 320000
_DF = 128
_DS = 58
_H = 64
_ALPHA = 0.4
_THETA = 0.9

_NC = 2
_NS = 16
_NW = _NC * _NS
_K = 125
_NCHUNK = _E // _K
_CPT = _NCHUNK // _NS
_CHD = _NCHUNK // _NW
_RPS = 640
_TAIL = _N - 15 * _RPS
_ZR = 128

_RB = 1000


def _mesh():
    return plsc.VectorSubcoreMesh(core_axis_name="c", subcore_axis_name="s")


def _zero_slab(s, sp_ref, zbuf):
    lo = s * _RPS
    for t in range(_RPS // _ZR):
        @pl.when(jnp.logical_or(s < 15, t < _TAIL // _ZR))
        def _():
            pltpu.sync_copy(zbuf, sp_ref.at[pl.ds(lo + t * _ZR, _ZR)])

    @pl.when(s == 15)
    def _():
        pltpu.sync_copy(zbuf.at[pl.ds(0, _TAIL % _ZR)],
                        sp_ref.at[pl.ds(15 * _RPS + (_TAIL // _ZR) * _ZR,
                                        _TAIL % _ZR)])


def _dump(c, s, sp_ref, out):
    @pl.when(s < 15)
    def _():
        pltpu.sync_copy(sp_ref.at[pl.ds(s * _RPS, _RPS)],
                        out.at[c, pl.ds(s * _RPS, _RPS)])

    @pl.when(s == 15)
    def _():
        pltpu.sync_copy(sp_ref.at[pl.ds(15 * _RPS, _TAIL)],
                        out.at[c, pl.ds(15 * _RPS, _TAIL)])


def _deg_body(col2d, degp, col_v, ones_v, zbuf, deg_sp, ssa):
    c = lax.axis_index("c")
    s = lax.axis_index("s")
    w = c * _NS + s

    def fill_ones(i, carry):
        for t in range(_H // 16):
            ones_v[i, pl.ds(16 * t, 16)] = jnp.ones((16,), jnp.float32)
        return carry

    lax.fori_loop(0, _K, fill_ones, 0)

    def fill_z(i, carry):
        for t in range(_H // 16):
            zbuf[i, pl.ds(16 * t, 16)] = jnp.zeros((16,), jnp.float32)
        return carry

    lax.fori_loop(0, _ZR, fill_z, 0)

    _zero_slab(s, deg_sp, zbuf)
    pltpu.sync_copy(col2d.at[pl.ds(w * _CHD, _CHD)], col_v)
    plsc.subcore_barrier()

    def chunk(j, carry):
        pltpu.async_copy(ones_v, deg_sp.at[col_v.at[j]], ssa, add=True)

        @pl.when(j >= 8)
        def _():
            pltpu.make_async_copy(ones_v, deg_sp.at[col_v.at[j - 8]], ssa).wait()

        return carry

    lax.fori_loop(0, _CHD, chunk, 0)

    def drain(j, carry):
        pltpu.make_async_copy(ones_v, deg_sp.at[col_v.at[j]], ssa).wait()
        return carry

    lax.fori_loop(_CHD - 8, _CHD, drain, 0)
    plsc.subcore_barrier()
    _dump(c, s, deg_sp, degp)


_sc_deg = pl.kernel(
    _deg_body,
    out_type=jax.ShapeDtypeStruct((_NC, _N, _H), jnp.float32),
    mesh=_mesh(),
    scratch_types=[
        pltpu.VMEM((_CHD, _K), jnp.int32),
        pltpu.VMEM((_K, _H), jnp.float32),
        pltpu.VMEM((_ZR, _H), jnp.float32),
        pltpu.VMEM_SHARED((_N, _H), jnp.float32),
        pltpu.SemaphoreType.DMA,
    ],
    compiler_params=pltpu.CompilerParams(use_tc_tiling_on_sc=False),
    name="sc_deg_hist",
)


def _mp_body(h2, row2d, col2d, out, row_v, col_v, rba, rbb, zbuf, agg_sp,
             gsa, gsb, ssa, ssb):
    c = lax.axis_index("c")
    s = lax.axis_index("s")

    def fill_z(i, carry):
        for t in range(_H // 16):
            zbuf[i, pl.ds(16 * t, 16)] = jnp.zeros((16,), jnp.float32)
        return carry

    lax.fori_loop(0, _ZR, fill_z, 0)
    _zero_slab(s, agg_sp, zbuf)
    pltpu.sync_copy(row2d.at[pl.ds(s * _CPT, _CPT)], row_v)
    pltpu.sync_copy(col2d.at[pl.ds(s * _CPT, _CPT)], col_v)
    plsc.subcore_barrier()

    hb = h2.at[c]

    pltpu.async_copy(hb.at[row_v.at[0]], rba, gsa)

    def chunk(i, carry):
        j0 = 2 * i
        pltpu.make_async_copy(hb.at[row_v.at[j0]], rba, gsa).wait()

        @pl.when(i > 0)
        def _():
            pltpu.make_async_copy(rbb, agg_sp.at[col_v.at[j0 - 1]], ssb).wait()

        pltpu.async_copy(hb.at[row_v.at[j0 + 1]], rbb, gsb)
        pltpu.async_copy(rba, agg_sp.at[col_v.at[j0]], ssa, add=True)
        pltpu.make_async_copy(hb.at[row_v.at[j0 + 1]], rbb, gsb).wait()
        pltpu.make_async_copy(rba, agg_sp.at[col_v.at[j0]], ssa).wait()

        @pl.when(i < _CPT // 2 - 1)
        def _():
            pltpu.async_copy(hb.at[row_v.at[j0 + 2]], rba, gsa)

        pltpu.async_copy(rbb, agg_sp.at[col_v.at[j0 + 1]], ssb, add=True)
        return carry

    lax.fori_loop(0, _CPT // 2, chunk, 0)
    pltpu.make_async_copy(rbb, agg_sp.at[col_v.at[_CPT - 1]], ssb).wait()
    plsc.subcore_barrier()
    _dump(c, s, agg_sp, out)


_sc_mp = pl.kernel(
    _mp_body,
    out_type=jax.ShapeDtypeStruct((_NC, _N, _H), jnp.float32),
    mesh=_mesh(),
    scratch_types=[
        pltpu.VMEM((_CPT, _K), jnp.int32),
        pltpu.VMEM((_CPT, _K), jnp.int32),
        pltpu.VMEM((_K, _H), jnp.float32),
        pltpu.VMEM((_K, _H), jnp.float32),
        pltpu.VMEM((_ZR, _H), jnp.float32),
        pltpu.VMEM_SHARED((_N, _H), jnp.float32),
        pltpu.SemaphoreType.DMA,
        pltpu.SemaphoreType.DMA,
        pltpu.SemaphoreType.DMA,
        pltpu.SemaphoreType.DMA,
    ],
    compiler_params=pltpu.CompilerParams(use_tc_tiling_on_sc=False),
    name="sc_mp_round",
)


def _pre_body(dgp, x2, ds2, w0b, b0b, w11b, b11b, h128, h1128, hsb, dis):
    deg = dgp[0] + dgp[1]
    d = jnp.where(deg > 0, lax.rsqrt(jnp.maximum(deg, 1e-12)), 0.0)
    a = jnp.maximum(
        jnp.dot(x2[...], w0b[...], preferred_element_type=jnp.float32) + b0b[...], 0.0)
    b = jnp.maximum(
        jnp.dot(ds2[...], w11b[...], preferred_element_type=jnp.float32) + b11b[...], 0.0)
    h128[...] = a
    h1128[...] = b
    hsb[...] = d * jnp.stack([a, b], axis=0)
    dis[...] = d


_pre = pl.pallas_call(
    _pre_body,
    grid=(_N2 // _RB,),
    in_specs=[
        pl.BlockSpec((_NC, _RB, 128), lambda i: (0, i, 0)),
        pl.BlockSpec((_RB, 2 * _DF), lambda i: (i, 0)),
        pl.BlockSpec((_RB, 128), lambda i: (i, 0)),
        pl.BlockSpec((2 * _DF, 128), lambda i: (0, 0)),
        pl.BlockSpec((1, 128), lambda i: (0, 0)),
        pl.BlockSpec((128, 128), lambda i: (0, 0)),
        pl.BlockSpec((1, 128), lambda i: (0, 0)),
    ],
    out_specs=[
        pl.BlockSpec((_RB, 128), lambda i: (i, 0)),
        pl.BlockSpec((_RB, 128), lambda i: (i, 0)),
        pl.BlockSpec((_NC, _RB, 128), lambda i: (0, i, 0)),
        pl.BlockSpec((_RB, 128), lambda i: (i, 0)),
    ],
    out_shape=[
        jax.ShapeDtypeStruct((_N2, 128), jnp.float32),
        jax.ShapeDtypeStruct((_N2, 128), jnp.float32),
        jax.ShapeDtypeStruct((_NC, _N2, 128), jnp.float32),
        jax.ShapeDtypeStruct((_N2, 128), jnp.float32),
    ],
    name="tc_pre",
)


def _mix(beta, p, dis, h0, h10, w1a, w2a, w1b, w2b):
    d = dis[...]
    oa = _mix_half(beta, (1.0 - _ALPHA) * (d * p[0]), _ALPHA * h0[...], w1a, w2a)
    ob = _mix_half(beta, (1.0 - _ALPHA) * (d * p[1]), _ALPHA * h10[...], w1b, w2b)
    return oa, ob, d


def _mix_half(beta, aggh, h0a, w1, w2):
    o = (1.0 - beta) * aggh + beta * jnp.dot(
        aggh, w1[...], preferred_element_type=jnp.float32)
    o = o + (1.0 - beta) * h0a + beta * jnp.dot(
        h0a, w2[...], preferred_element_type=jnp.float32)
    return jnp.maximum(o, 0.0)


def _comb_mid_body(beta, p, dis, h0, h10, w1a, w2a, w1b, w2b, hsb):
    oa, ob, d = _mix(beta, p, dis, h0, h10, w1a, w2a, w1b, w2b)
    hsb[...] = d * jnp.stack([oa, ob], axis=0)


def _comb_last_body(beta, p, dis, h0, h10, w1a, w2a, w1b, w2b,
                    lwa, lwb, lb2, z2):
    oa, ob, _ = _mix(beta, p, dis, h0, h10, w1a, w2a, w1b, w2b)
    za = jnp.dot(oa, lwa[...], preferred_element_type=jnp.float32)
    zb = jnp.dot(ob, lwb[...], preferred_element_type=jnp.float32)
    z2[...] = jnp.stack([za, zb], axis=0) + lb2[...]


_W_SPECS = [
    pl.BlockSpec((_NC, _RB, 128), lambda i: (0, i, 0)),
    pl.BlockSpec((_RB, 128), lambda i: (i, 0)),
    pl.BlockSpec((_RB, 128), lambda i: (i, 0)),
    pl.BlockSpec((_RB, 128), lambda i: (i, 0)),
    pl.BlockSpec((128, 128), lambda i: (0, 0)),
    pl.BlockSpec((128, 128), lambda i: (0, 0)),
    pl.BlockSpec((128, 128), lambda i: (0, 0)),
    pl.BlockSpec((128, 128), lambda i: (0, 0)),
]

_comb_mid0 = pl.pallas_call(
    functools.partial(_comb_mid_body, float(np.log(_THETA / 1.0 + 1.0))),
    grid=(_N2 // _RB,),
    in_specs=_W_SPECS,
    out_specs=[pl.BlockSpec((_NC, _RB, 128), lambda i: (0, i, 0))],
    out_shape=[jax.ShapeDtypeStruct((_NC, _N2, 128), jnp.float32)],
    name="tc_combine_mid",
)

_comb_last1 = pl.pallas_call(
    functools.partial(_comb_last_body, float(np.log(_THETA / 2.0 + 1.0))),
    grid=(_N2 // _RB,),
    in_specs=_W_SPECS + [
        pl.BlockSpec((128, 2), lambda i: (0, 0)),
        pl.BlockSpec((128, 2), lambda i: (0, 0)),
        pl.BlockSpec((_NC, 1, 2), lambda i: (0, 0, 0)),
    ],
    out_specs=[pl.BlockSpec((_NC, _RB, 2), lambda i: (0, i, 0))],
    out_shape=[jax.ShapeDtypeStruct((_NC, _N2, 2), jnp.float32)],
    name="tc_combine_last",
)


def _blkdiag(w):
    fi, fo = w.shape
    zz = jnp.zeros((fi, fo), jnp.float32)
    return jnp.concatenate([
        jnp.concatenate([w, zz], axis=1),
        jnp.concatenate([zz, w], axis=1),
    ], axis=0)


def kernel(x, data_str, edge_index, lins0_w, lins0_b, lins1_w, lins1_b,
           lin11_w, lin11_b, lin3_w, lin3_b,
           convs_w1, convs_w2, convs1_w1, convs1_w2):
    row2d = edge_index[0].reshape(_NCHUNK, _K)
    col2d = edge_index[1].reshape(_NCHUNK, _K)

    x2 = x.reshape(_N2, 2 * _DF)
    ds64 = jnp.pad(data_str, ((0, 0), (0, _H - _DS)))
    ds2 = ds64.reshape(_N2, 128)
    w11p = jnp.pad(lin11_w, ((0, _H - _DS), (0, 0)))
    b2 = jnp.concatenate([lins0_b, lins0_b]).reshape(1, 128)
    b112 = jnp.concatenate([lin11_b, lin11_b]).reshape(1, 128)

    degp = _sc_deg(col2d)
    h, h1, hsb, dis = _pre(degp.reshape(_NC, _N2, 128), x2, ds2,
                           _blkdiag(lins0_w), b2, _blkdiag(w11p), b112)

    p = _sc_mp(hsb.reshape(_NC, _N, _H), row2d, col2d)
    (hsb,) = _comb_mid0(p.reshape(_NC, _N2, 128), dis, h, h1,
                        _blkdiag(convs_w1[0]), _blkdiag(convs_w2[0]),
                        _blkdiag(convs1_w1[0]), _blkdiag(convs1_w2[0]))
    p = _sc_mp(hsb.reshape(_NC, _N, _H), row2d, col2d)
    lb2 = jnp.stack([jnp.broadcast_to(lins1_b, (2,)),
                     jnp.broadcast_to(lin3_b, (2,))]).reshape(_NC, 1, 2)
    (z2,) = _comb_last1(p.reshape(_NC, _N2, 128), dis, h, h1,
                        _blkdiag(convs_w1[1]), _blkdiag(convs_w2[1]),
                        _blkdiag(convs1_w1[1]), _blkdiag(convs1_w2[1]),
                        _blkdiag(lins1_w), _blkdiag(lin3_w), lb2)
    z = z2[0].reshape(_N, 1)
    z1 = z2[1].reshape(_N, 1)
    return (z, z1)
